# Initial kernel scaffold; baseline (speedup 1.0000x reference)
import jax
import jax.numpy as jnp
from jax.experimental import pallas as pl


def kernel(data_encoded, data_decoded, data_true, label_true, batch_size):
    raise NotImplementedError("write your pallas kernel here")


# speedup vs baseline: 181.0372x; 181.0372x over previous
import numpy as np
import jax
import jax.numpy as jnp
from jax.experimental import pallas as pl
from jax.experimental.pallas import tpu as pltpu

_B = 16384
_CAT_SLICES = [(1, 10), (12, 29), (30, 33), (33, 40), (40, 64), (64, 79),
               (79, 84), (84, 94), (94, 96), (96, 99), (99, 105), (105, 113),
               (116, 122), (122, 128), (128, 151), (151, 159), (160, 165)]
_CONT_COLS = [0, 10, 11, 29, 113, 114, 115, 159, 165, 166, 167]
_NBINS = 800
_NHI = 25
_NLO = 32
_EPS = 1e-10
_NB = 8
_R = _B // _NB


def _seg_matrix():
    s = np.zeros((168, len(_CAT_SLICES)), dtype=np.float32)
    for j, (a, b) in enumerate(_CAT_SLICES):
        s[a:b, j] = 1.0
    return s


def _cont_mask():
    m = np.zeros((1, 168), dtype=np.float32)
    m[0, _CONT_COLS] = 1.0
    return m


def _cat_mask():
    m = np.zeros((1, 168), dtype=np.float32)
    for (a, b) in _CAT_SLICES:
        m[0, a:b] = 1.0
    return m


def _loss_kernel(dec_ref, true_ref, enc_ref, lab_ref, seg_ref, cmask_ref,
                 kmask_ref, out_ref, smem_acc, mn_ref, mx_ref, hm_ref, hf_ref):
    p = pl.program_id(0)
    j = pl.program_id(1)

    @pl.when(jnp.logical_and(p == 0, j == 0))
    def _init():
        smem_acc[0] = 0.0
        smem_acc[1] = 0.0
        smem_acc[2] = 0.0
        hm_ref[...] = jnp.zeros_like(hm_ref)
        hf_ref[...] = jnp.zeros_like(hf_ref)

    @pl.when(p == 0)
    def _phase0():
        dec = dec_ref[...]
        true = true_ref[...]
        diff = dec - true
        mse_part = jnp.sum(diff * diff * cmask_ref[...])

        e = jnp.exp(dec)
        sumexp = jax.lax.dot_general(e, seg_ref[...], (((1,), (0,)), ((), ())),
                                     preferred_element_type=jnp.float32)
        ce_part = jnp.sum(jnp.log(sumexp)) - jnp.sum(dec * true * kmask_ref[...])

        nf_part = jnp.sum(lab_ref[...])

        smem_acc[0] += mse_part
        smem_acc[1] += ce_part
        smem_acc[2] += nf_part

        enc = enc_ref[...]
        bmn = jnp.min(enc, axis=1, keepdims=True)
        bmx = jnp.max(enc, axis=1, keepdims=True)

        @pl.when(j == 0)
        def _():
            mn_ref[...] = bmn
            mx_ref[...] = bmx

        @pl.when(j > 0)
        def _():
            mn_ref[...] = jnp.minimum(mn_ref[...], bmn)
            mx_ref[...] = jnp.maximum(mx_ref[...], bmx)

    @pl.when(p == 1)
    def _phase1():
        lab1 = lab_ref[...]
        for c in range(10):
            v = enc_ref[c:c + 1, :]
            mn = mn_ref[c, 0]
            mx = mx_ref[c, 0]
            invw = jnp.float32(_NBINS) / (mx - mn)
            b0 = jnp.clip(((v - mn) * invw).astype(jnp.int32), 0, _NBINS - 1)
            bf = b0.astype(jnp.float32)
            t0 = bf * jnp.float32(1.0 / _NBINS)
            t1 = (bf + 1.0) * jnp.float32(1.0 / _NBINS)
            e0 = mn * (1.0 - t0) + mx * t0
            e1 = mn * (1.0 - t1) + mx * t1
            b = b0 + (v >= e1).astype(jnp.int32) - (v < e0).astype(jnp.int32)
            b = jnp.clip(b, 0, _NBINS - 1)

            hi = jax.lax.shift_right_logical(b, 5)
            lo = b - jax.lax.shift_left(hi, 5)
            oh_hi = (hi == jax.lax.broadcasted_iota(
                jnp.int32, (_NHI, _R), 0)).astype(jnp.float32)
            oh_lo = (lo == jax.lax.broadcasted_iota(
                jnp.int32, (_NLO, _R), 0)).astype(jnp.float32)
            a_f = oh_hi * lab1
            a_m = oh_hi - a_f
            h_m = jax.lax.dot_general(a_m, oh_lo, (((1,), (1,)), ((), ())),
                                      preferred_element_type=jnp.float32)
            h_f = jax.lax.dot_general(a_f, oh_lo, (((1,), (1,)), ((), ())),
                                      preferred_element_type=jnp.float32)
            hm_ref[c] += h_m
            hf_ref[c] += h_f

    @pl.when(jnp.logical_and(p == 1, j == _NB - 1))
    def _final():
        n_f = smem_acc[2]
        n_m = jnp.float32(_B) - n_f
        kl = jnp.float32(0.0)
        for c in range(10):
            pp = hm_ref[c] / n_m
            qq = hf_ref[c] / n_f
            mm = 0.5 * (pp + qq)
            kl += jnp.sum(pp * jnp.log((pp + _EPS) / (mm + _EPS)))
            kl += jnp.sum(qq * jnp.log((qq + _EPS) / (mm + _EPS)))
        kld = 0.5 * kl

        inv_b = jnp.float32(1.0 / _B)
        mse_loss = smem_acc[0] * inv_b
        ce_loss = smem_acc[1] * inv_b
        ajsd = 0.5 * kld
        multi = 0.5 * (mse_loss + ce_loss) + ajsd

        lane = jax.lax.broadcasted_iota(jnp.int32, (1, 128), 1)
        out_ref[...] = jnp.where(lane == 0, multi,
                       jnp.where(lane == 1, mse_loss,
                       jnp.where(lane == 2, ce_loss, ajsd)))


def kernel(data_encoded, data_decoded, data_true, label_true, batch_size):
    seg = jnp.asarray(_seg_matrix())
    cmask = jnp.asarray(_cont_mask())
    kmask = jnp.asarray(_cat_mask())
    enc_t = data_encoded.T
    lab1 = label_true[:, 1].reshape(1, _B)

    out = pl.pallas_call(
        _loss_kernel,
        grid=(2, _NB),
        in_specs=[
            pl.BlockSpec((_R, 168), lambda p, j: (jnp.where(p == 0, j, 0), 0)),
            pl.BlockSpec((_R, 168), lambda p, j: (jnp.where(p == 0, j, 0), 0)),
            pl.BlockSpec((10, _R), lambda p, j: (0, j)),
            pl.BlockSpec((1, _R), lambda p, j: (0, j)),
            pl.BlockSpec((168, 17), lambda p, j: (0, 0)),
            pl.BlockSpec((1, 168), lambda p, j: (0, 0)),
            pl.BlockSpec((1, 168), lambda p, j: (0, 0)),
        ],
        out_specs=pl.BlockSpec((1, 128), lambda p, j: (0, 0)),
        out_shape=jax.ShapeDtypeStruct((1, 128), jnp.float32),
        scratch_shapes=[
            pltpu.SMEM((4,), jnp.float32),
            pltpu.VMEM((10, 1), jnp.float32),
            pltpu.VMEM((10, 1), jnp.float32),
            pltpu.VMEM((10, _NHI, _NLO), jnp.float32),
            pltpu.VMEM((10, _NHI, _NLO), jnp.float32),
        ],
        compiler_params=pltpu.CompilerParams(
            dimension_semantics=("arbitrary", "arbitrary"),
        ),
    )(data_decoded, data_true, enc_t, lab1, seg, cmask, kmask)
    return (out[0, 0], out[0, 1], out[0, 2], out[0, 3])
